# baseline (speedup 1.0000x reference)
import jax
import jax.numpy as jnp
from jax import lax
from jax.experimental import pallas as pl
from jax.experimental.pallas import tpu as pltpu
from jax.experimental.pallas import tpu_sc as plsc

B, L, D = 4, 2048, 2048
NC, NS = 2, 16
NW = NC * NS
B2 = 2
NBANDS = NW // B2
RPW = L // NBANDS
CH = 8
NCH = RPW // CH
UNROLL = 4


def _x_copy(x_hbm, xb, sx, ci, k, b0, base):
    row0 = base + ci * CH
    return pltpu.make_async_copy(
        x_hbm.at[pl.ds(b0, B2), pl.ds(row0, CH)], xb.at[k], sx.at[k]
    )


def _t_copy(t_hbm, tb, st, ci, base):
    row0 = base + ci * CH
    return pltpu.make_async_copy(t_hbm.at[pl.ds(row0, CH)], tb, st)


def _o_copy(o_hbm, xb, so, ci, k, b0, base):
    row0 = base + ci * CH
    return pltpu.make_async_copy(
        xb.at[k], o_hbm.at[pl.ds(b0, B2), pl.ds(row0, CH)], so.at[k]
    )


def _sc_body(x_hbm, t_hbm, o_hbm, xb, tb, sx, st, so):
    c = lax.axis_index("c")
    s = lax.axis_index("s")
    wid = s * NC + c
    b0 = (wid % B2) * B2
    base = (wid // B2) * RPW

    _t_copy(t_hbm, tb, st, 0, base).start()
    _x_copy(x_hbm, xb, sx, 0, 0, b0, base).start()

    for ci in range(NCH):
        k = ci % 3
        kn = (ci + 1) % 3
        if ci >= 2:
            _o_copy(o_hbm, xb, so, ci - 2, kn, b0, base).wait()
        if ci + 1 < NCH:
            _x_copy(x_hbm, xb, sx, ci + 1, kn, b0, base).start()
        _t_copy(t_hbm, tb, st, ci, base).wait()
        _x_copy(x_hbm, xb, sx, ci, k, b0, base).wait()

        @plsc.parallel_loop(0, CH * D, step=16, unroll=UNROLL)
        def _(g):
            i = g // D
            cc = g % D
            tv = tb[i, pl.ds(cc, 16)]
            for b in range(B2):
                xb[k, b, i, pl.ds(cc, 16)] = xb[k, b, i, pl.ds(cc, 16)] + tv

        _o_copy(o_hbm, xb, so, ci, k, b0, base).start()
        if ci + 1 < NCH:
            _t_copy(t_hbm, tb, st, ci + 1, base).start()

    for ci in (NCH - 2, NCH - 1):
        _o_copy(o_hbm, xb, so, ci, ci % 3, b0, base).wait()


def kernel(x, table):
    mesh = plsc.VectorSubcoreMesh(
        core_axis_name="c", subcore_axis_name="s", num_cores=NC, num_subcores=NS
    )
    return pl.kernel(
        _sc_body,
        mesh=mesh,
        out_type=jax.ShapeDtypeStruct((B, L, D), jnp.float32),
        scratch_types=[
            pltpu.VMEM((3, B2, CH, D), jnp.float32),
            pltpu.VMEM((CH, D), jnp.float32),
            pltpu.SemaphoreType.DMA((3,)),
            pltpu.SemaphoreType.DMA,
            pltpu.SemaphoreType.DMA((3,)),
        ],
    )(x, table)


# speedup vs baseline: 1.2317x; 1.2317x over previous
import jax
import jax.numpy as jnp
from jax import lax
from jax.experimental import pallas as pl
from jax.experimental.pallas import tpu as pltpu
from jax.experimental.pallas import tpu_sc as plsc

B, L, D = 4, 2048, 2048
NC, NS = 2, 16
NW = NC * NS
RPW = L // NW
CH = 4
NCH = RPW // CH
UNROLL = 4


def _x_copy(x_hbm, xb, sx, ci, k, base):
    row0 = base + ci * CH
    return pltpu.make_async_copy(
        x_hbm.at[:, pl.ds(row0, CH)], xb.at[k], sx.at[k]
    )


def _t_copy(t_hbm, tb, st, ci, base):
    row0 = base + ci * CH
    return pltpu.make_async_copy(
        t_hbm.at[pl.ds(row0, CH)], tb.at[ci % 2], st.at[ci % 2]
    )


def _o_copy(o_hbm, xb, so, ci, k, base):
    row0 = base + ci * CH
    return pltpu.make_async_copy(
        xb.at[k], o_hbm.at[:, pl.ds(row0, CH)], so.at[k]
    )


def _sc_body(x_hbm, t_hbm, o_hbm, xb, tb, sx, st, so):
    c = lax.axis_index("c")
    s = lax.axis_index("s")
    wid = s * NC + c
    base = wid * RPW

    _t_copy(t_hbm, tb, st, 0, base).start()
    _x_copy(x_hbm, xb, sx, 0, 0, base).start()
    _t_copy(t_hbm, tb, st, 1, base).start()

    for ci in range(NCH):
        k = ci % 3
        kn = (ci + 1) % 3
        if ci >= 2:
            _o_copy(o_hbm, xb, so, ci - 2, kn, base).wait()
        if ci + 1 < NCH:
            _x_copy(x_hbm, xb, sx, ci + 1, kn, base).start()
        _t_copy(t_hbm, tb, st, ci, base).wait()
        _x_copy(x_hbm, xb, sx, ci, k, base).wait()

        tk = ci % 2

        @plsc.parallel_loop(0, CH * D, step=16, unroll=UNROLL)
        def _(g):
            i = g // D
            cc = g % D
            tv = tb[tk, i, pl.ds(cc, 16)]
            for b in range(B):
                xb[k, b, i, pl.ds(cc, 16)] = xb[k, b, i, pl.ds(cc, 16)] + tv

        _o_copy(o_hbm, xb, so, ci, k, base).start()
        if ci + 2 < NCH:
            _t_copy(t_hbm, tb, st, ci + 2, base).start()

    for ci in (NCH - 2, NCH - 1):
        _o_copy(o_hbm, xb, so, ci, ci % 3, base).wait()


def kernel(x, table):
    mesh = plsc.VectorSubcoreMesh(
        core_axis_name="c", subcore_axis_name="s", num_cores=NC, num_subcores=NS
    )
    return pl.kernel(
        _sc_body,
        mesh=mesh,
        out_type=jax.ShapeDtypeStruct((B, L, D), jnp.float32),
        scratch_types=[
            pltpu.VMEM((3, B, CH, D), jnp.float32),
            pltpu.VMEM((2, CH, D), jnp.float32),
            pltpu.SemaphoreType.DMA((3,)),
            pltpu.SemaphoreType.DMA((2,)),
            pltpu.SemaphoreType.DMA((3,)),
        ],
    )(x, table)
